# Initial kernel scaffold; baseline (speedup 1.0000x reference)
import jax
import jax.numpy as jnp
from jax.experimental import pallas as pl


def kernel(node, edge_index, W, a):
    raise NotImplementedError("write your pallas kernel here")


# speedup vs baseline: 12.1844x; 12.1844x over previous
import functools

import jax
import jax.numpy as jnp
from jax import lax
from jax.experimental import pallas as pl
from jax.experimental.pallas import tpu as pltpu
from jax.experimental.pallas import tpu_sc as plsc

_ALPHA = 0.2
_L = 16
_C = 128


def _prep_body(node_ref, wt_ref, a1_ref, a2_ref, tnh_ref, s1_ref, s2_ref):
    tn = jnp.dot(node_ref[...], wt_ref[...], preferred_element_type=jnp.float32)
    dh = tn.shape[1] // 2
    tnh_ref[...] = jnp.stack([tn[:, :dh], tn[:, dh:]])
    s1_ref[...] = jnp.sum(tn * a1_ref[...], axis=1, keepdims=True)
    s2_ref[...] = jnp.sum(tn * a2_ref[...], axis=1, keepdims=True)


def _combine_body(p_ref, d_ref, o_ref):
    den = d_ref[...] + 1e-10
    o_ref[...] = jnp.concatenate([p_ref[0], p_ref[1]], axis=1) / den[:, None]


def _sc_body(n_edges, chunks_per_sub,
             tnh_hbm, s1_hbm, s2_hbm, src_hbm, tgt_hbm,
             pout_hbm, pden_hbm,
             srcv, tgtv, wv, s1v, s2v, rows, zbuf, acc, den):
    dh = rows.shape[1]
    nr = acc.shape[0]
    rows_per_sub = nr // 16
    den_per_sub = nr // 10
    cid = lax.axis_index("c")
    sid = lax.axis_index("s")

    pltpu.sync_copy(src_hbm.at[sid], srcv)
    pltpu.sync_copy(tgt_hbm.at[sid], tgtv)
    pltpu.sync_copy(s1_hbm, s1v)
    pltpu.sync_copy(s2_hbm, s2v)

    z16 = jnp.zeros((_L,), jnp.float32)

    def _zero_rows(r, _):
        for k in range(dh // _L):
            rows[r, pl.ds(k * _L, _L)] = z16
        return _
    lax.fori_loop(0, _C, _zero_rows, None)
    for k in range(zbuf.shape[0] // _L):
        zbuf[pl.ds(k * _L, _L)] = z16
    q = rows_per_sub // 5
    for b in range(5):
        pltpu.sync_copy(rows.at[pl.ds(0, q), :],
                        acc.at[pl.ds(sid * rows_per_sub + b * q, q), :])

    @pl.when(sid < 10)
    def _():
        pltpu.sync_copy(zbuf.at[pl.ds(0, den_per_sub)],
                        den.at[pl.ds(sid * den_per_sub, den_per_sub)])
    plsc.subcore_barrier()

    lane = lax.broadcasted_iota(jnp.int32, (_L,), 0)
    base_e = sid * (chunks_per_sub * _C)

    def _chunk(j, _):
        for v in range(_C // _L):
            isrc = srcv[j, pl.ds(v * _L, _L)]
            itgt = tgtv[j, pl.ds(v * _L, _L)]
            x = (plsc.load_gather(s1v, [isrc >> 7, isrc & 127])
                 + plsc.load_gather(s2v, [itgt >> 7, itgt & 127]))
            w = jnp.exp(jnp.maximum(x, _ALPHA * x))
            eid = base_e + j * _C + (v * _L) + lane
            w = jnp.where(eid < n_edges, w, 0.0)
            wv[j, pl.ds(v * _L, _L)] = w
        pltpu.sync_copy(tnh_hbm.at[cid].at[tgtv.at[j]], rows)

        def _scale(g, _):
            wvec = wv[j, pl.ds(g * _L, _L)]
            for t in range(_L):
                i = g * _L + t
                ws = wvec[t]
                for k in range(dh // _L):
                    sl = pl.ds(k * _L, _L)
                    rows[i, sl] = rows[i, sl] * ws
            return _
        lax.fori_loop(0, _C // _L, _scale, None)
        pltpu.sync_copy(rows, acc.at[srcv.at[j]], add=True)
        pltpu.sync_copy(wv.at[j], den.at[srcv.at[j]], add=True)
        return _
    lax.fori_loop(0, chunks_per_sub, _chunk, None)
    plsc.subcore_barrier()

    r0 = sid * rows_per_sub
    pltpu.sync_copy(acc.at[pl.ds(r0, rows_per_sub), :],
                    pout_hbm.at[cid, pl.ds(r0, rows_per_sub), :])

    @pl.when((cid == 0) & (sid < 10))
    def _():
        pltpu.sync_copy(den.at[pl.ds(sid * den_per_sub, den_per_sub)],
                        pden_hbm.at[pl.ds(sid * den_per_sub, den_per_sub)])


def kernel(node, edge_index, W, a):
    n, din = node.shape
    dout = W.shape[0]
    dh = dout // 2
    e = edge_index.shape[1]

    bn = 2000
    grid = n // bn
    tnh, s1, s2 = pl.pallas_call(
        _prep_body,
        grid=(grid,),
        in_specs=[
            pl.BlockSpec((bn, din), lambda i: (i, 0)),
            pl.BlockSpec((din, dout), lambda i: (0, 0)),
            pl.BlockSpec((1, dout), lambda i: (0, 0)),
            pl.BlockSpec((1, dout), lambda i: (0, 0)),
        ],
        out_specs=[
            pl.BlockSpec((2, bn, dh), lambda i: (0, i, 0)),
            pl.BlockSpec((bn, 1), lambda i: (i, 0)),
            pl.BlockSpec((bn, 1), lambda i: (i, 0)),
        ],
        out_shape=[
            jax.ShapeDtypeStruct((2, n, dh), jnp.float32),
            jax.ShapeDtypeStruct((n, 1), jnp.float32),
            jax.ShapeDtypeStruct((n, 1), jnp.float32),
        ],
    )(node, W.T, a[:, :dout], a[:, dout:])
    s1 = s1[:, 0]
    s2 = s2[:, 0]

    n_subs = 16
    per_s = -(-e // n_subs)
    per_s = -(-per_s // _C) * _C
    epad = n_subs * per_s
    pad = epad - e
    pad_idx = (jnp.arange(pad, dtype=jnp.int32) * 37) % n
    src = jnp.concatenate([edge_index[0], pad_idx]).reshape(n_subs, per_s // _C, _C)
    tgt = jnp.concatenate([edge_index[1], pad_idx]).reshape(n_subs, per_s // _C, _C)

    srows = -(-n // 128) * 128
    mesh = plsc.VectorSubcoreMesh(core_axis_name="c", subcore_axis_name="s")
    sc = pl.kernel(
        functools.partial(_sc_body, e, per_s // _C),
        out_type=[
            jax.ShapeDtypeStruct((2, n, dh), jnp.float32),
            jax.ShapeDtypeStruct((n,), jnp.float32),
        ],
        mesh=mesh,
        compiler_params=pltpu.CompilerParams(needs_layout_passes=False,
                                             use_tc_tiling_on_sc=False),
        scratch_types=[
            pltpu.VMEM((per_s // _C, _C), jnp.int32),
            pltpu.VMEM((per_s // _C, _C), jnp.int32),
            pltpu.VMEM((per_s // _C, _C), jnp.float32),
            pltpu.VMEM((srows // 128, 128), jnp.float32),
            pltpu.VMEM((srows // 128, 128), jnp.float32),
            pltpu.VMEM((_C, dh), jnp.float32),
            pltpu.VMEM((-(-(n // 10) // _L) * _L,), jnp.float32),
            pltpu.VMEM_SHARED((n, dh), jnp.float32),
            pltpu.VMEM_SHARED((n,), jnp.float32),
        ],
    )
    s1p = jnp.pad(s1, (0, srows - n)).reshape(srows // 128, 128)
    s2p = jnp.pad(s2, (0, srows - n)).reshape(srows // 128, 128)
    pout, pden = sc(tnh, s1p, s2p, src, tgt)

    bo = 2048
    go = -(-n // bo)
    out = pl.pallas_call(
        _combine_body,
        grid=(go,),
        in_specs=[
            pl.BlockSpec((2, bo, dh), lambda i: (0, i, 0)),
            pl.BlockSpec((bo,), lambda i: (i,)),
        ],
        out_specs=pl.BlockSpec((bo, dout), lambda i: (i, 0)),
        out_shape=jax.ShapeDtypeStruct((n, dout), jnp.float32),
    )(pout, pden)
    return out


# speedup vs baseline: 82.9956x; 6.8116x over previous
import functools

import jax
import jax.numpy as jnp
from jax import lax
from jax.experimental import pallas as pl
from jax.experimental.pallas import tpu as pltpu
from jax.experimental.pallas import tpu_sc as plsc

_ALPHA = 0.2
_L = 16
_C = 128


def _prep_body(node_ref, wt_ref, a1_ref, a2_ref, tnh_ref, s1_ref, s2_ref):
    tn = jnp.dot(node_ref[...], wt_ref[...], preferred_element_type=jnp.float32)
    dh = tn.shape[1] // 2
    tnh_ref[...] = jnp.stack([tn[:, :dh], tn[:, dh:]])
    s1_ref[...] = jnp.sum(tn * a1_ref[...], axis=1, keepdims=True)
    s2_ref[...] = jnp.sum(tn * a2_ref[...], axis=1, keepdims=True)


def _combine_body(p_ref, d_ref, o_ref):
    den = d_ref[...] + 1e-10
    o_ref[...] = jnp.concatenate([p_ref[0], p_ref[1]], axis=1) / den[:, None]


def _sc_body(n_edges, chunks_per_sub,
             tnh_hbm, s1_hbm, s2_hbm, src_hbm, tgt_hbm,
             pout_hbm, pden_hbm,
             srcv, tgtv, s1v, s2v,
             rows, zbuf, acc, den):
    dh = rows.shape[1]
    nr = acc.shape[0]
    rows_per_sub = nr // 16
    den_per_sub = nr // 10
    cid = lax.axis_index("c")
    sid = lax.axis_index("s")

    pltpu.sync_copy(src_hbm.at[sid], srcv)
    pltpu.sync_copy(tgt_hbm.at[sid], tgtv)
    pltpu.sync_copy(s1_hbm, s1v)
    pltpu.sync_copy(s2_hbm, s2v)

    z16 = jnp.zeros((_L,), jnp.float32)

    def _zero_rows(r, _):
        for k in range(dh // _L):
            rows[r, pl.ds(k * _L, _L)] = z16
        return _
    lax.fori_loop(0, _C, _zero_rows, None)
    for k in range(zbuf.shape[0] // _L):
        zbuf[pl.ds(k * _L, _L)] = z16
    q = rows_per_sub // 5
    for b in range(5):
        pltpu.sync_copy(rows.at[pl.ds(0, q), :],
                        acc.at[pl.ds(sid * rows_per_sub + b * q, q), :])

    @pl.when(sid < 10)
    def _():
        pltpu.sync_copy(zbuf.at[pl.ds(0, den_per_sub)],
                        den.at[pl.ds(sid * den_per_sub, den_per_sub)])
    plsc.subcore_barrier()

    lane = lax.broadcasted_iota(jnp.int32, (_L,), 0)
    base_e = sid * (chunks_per_sub * _C)

    @functools.partial(plsc.parallel_loop, 0, chunks_per_sub, unroll=2)
    def _chunk(j):
        def inner(buf, wbuf):
            pltpu.sync_copy(tnh_hbm.at[cid].at[tgtv.at[j]], buf)
            for v in range(_C // _L):
                isrc = srcv[j, pl.ds(v * _L, _L)]
                itgt = tgtv[j, pl.ds(v * _L, _L)]
                x = (plsc.load_gather(s1v, [isrc >> 7, isrc & 127])
                     + plsc.load_gather(s2v, [itgt >> 7, itgt & 127]))
                w = jnp.exp(jnp.maximum(x, _ALPHA * x))
                eid = base_e + j * _C + (v * _L) + lane
                w = jnp.where(eid < n_edges, w, 0.0)
                wbuf[pl.ds(v * _L, _L)] = w

            def _scale(gq, _):
                wvec = wbuf[pl.ds(gq * _L, _L)]
                for t in range(_L):
                    r = gq * _L + t
                    ws = wvec[t]
                    for k in range(dh // _L):
                        sl = pl.ds(k * _L, _L)
                        buf[r, sl] = buf[r, sl] * ws
                return _
            lax.fori_loop(0, _C // _L, _scale, None)
            pltpu.sync_copy(buf, acc.at[srcv.at[j]], add=True)
            pltpu.sync_copy(wbuf, den.at[srcv.at[j]], add=True)
        pl.run_scoped(inner, pltpu.VMEM((_C, dh), jnp.float32),
                      pltpu.VMEM((_C,), jnp.float32))
    plsc.subcore_barrier()

    r0 = sid * rows_per_sub
    pltpu.sync_copy(acc.at[pl.ds(r0, rows_per_sub), :],
                    pout_hbm.at[cid, pl.ds(r0, rows_per_sub), :])

    @pl.when((cid == 0) & (sid < 10))
    def _():
        pltpu.sync_copy(den.at[pl.ds(sid * den_per_sub, den_per_sub)],
                        pden_hbm.at[pl.ds(sid * den_per_sub, den_per_sub)])


def kernel(node, edge_index, W, a):
    n, din = node.shape
    dout = W.shape[0]
    dh = dout // 2
    e = edge_index.shape[1]

    bn = 2000
    grid = n // bn
    tnh, s1, s2 = pl.pallas_call(
        _prep_body,
        grid=(grid,),
        in_specs=[
            pl.BlockSpec((bn, din), lambda i: (i, 0)),
            pl.BlockSpec((din, dout), lambda i: (0, 0)),
            pl.BlockSpec((1, dout), lambda i: (0, 0)),
            pl.BlockSpec((1, dout), lambda i: (0, 0)),
        ],
        out_specs=[
            pl.BlockSpec((2, bn, dh), lambda i: (0, i, 0)),
            pl.BlockSpec((bn, 1), lambda i: (i, 0)),
            pl.BlockSpec((bn, 1), lambda i: (i, 0)),
        ],
        out_shape=[
            jax.ShapeDtypeStruct((2, n, dh), jnp.float32),
            jax.ShapeDtypeStruct((n, 1), jnp.float32),
            jax.ShapeDtypeStruct((n, 1), jnp.float32),
        ],
    )(node, W.T, a[:, :dout], a[:, dout:])
    s1 = s1[:, 0]
    s2 = s2[:, 0]

    n_subs = 16
    per_s = -(-e // n_subs)
    per_s = -(-per_s // _C) * _C
    epad = n_subs * per_s
    pad = epad - e
    pad_idx = (jnp.arange(pad, dtype=jnp.int32) * 37) % n
    src = jnp.concatenate([edge_index[0], pad_idx]).reshape(n_subs, per_s // _C, _C)
    tgt = jnp.concatenate([edge_index[1], pad_idx]).reshape(n_subs, per_s // _C, _C)

    srows = -(-n // 128) * 128
    mesh = plsc.VectorSubcoreMesh(core_axis_name="c", subcore_axis_name="s")
    sc = pl.kernel(
        functools.partial(_sc_body, e, per_s // _C),
        out_type=[
            jax.ShapeDtypeStruct((2, n, dh), jnp.float32),
            jax.ShapeDtypeStruct((n,), jnp.float32),
        ],
        mesh=mesh,
        compiler_params=pltpu.CompilerParams(needs_layout_passes=False,
                                             use_tc_tiling_on_sc=False),
        scratch_types=[
            pltpu.VMEM((per_s // _C, _C), jnp.int32),
            pltpu.VMEM((per_s // _C, _C), jnp.int32),
            pltpu.VMEM((srows // 128, 128), jnp.float32),
            pltpu.VMEM((srows // 128, 128), jnp.float32),
            pltpu.VMEM((_C, dh), jnp.float32),
            pltpu.VMEM((-(-(n // 10) // _L) * _L,), jnp.float32),
            pltpu.VMEM_SHARED((n, dh), jnp.float32),
            pltpu.VMEM_SHARED((n,), jnp.float32),
        ],
    )
    s1p = jnp.pad(s1, (0, srows - n)).reshape(srows // 128, 128)
    s2p = jnp.pad(s2, (0, srows - n)).reshape(srows // 128, 128)
    pout, pden = sc(tnh, s1p, s2p, src, tgt)

    bo = 2048
    go = -(-n // bo)
    out = pl.pallas_call(
        _combine_body,
        grid=(go,),
        in_specs=[
            pl.BlockSpec((2, bo, dh), lambda i: (0, i, 0)),
            pl.BlockSpec((bo,), lambda i: (i,)),
        ],
        out_specs=pl.BlockSpec((bo, dout), lambda i: (i, 0)),
        out_shape=jax.ShapeDtypeStruct((n, dout), jnp.float32),
    )(pout, pden)
    return out
